# Initial kernel scaffold; baseline (speedup 1.0000x reference)
import jax
import jax.numpy as jnp
from jax.experimental import pallas as pl


def kernel(img, trf):
    raise NotImplementedError("write your pallas kernel here")


# speedup vs baseline: 1.0163x; 1.0163x over previous
import functools

import jax
import jax.numpy as jnp
from jax import lax
from jax.experimental import pallas as pl
from jax.experimental.pallas import tpu as pltpu
from jax.experimental.pallas import tpu_sc as plsc

_B, _H, _W, _C = 4, 384, 384, 96
_HW = _H * _W
_N = _B * _HW
_NW = 32
_PPT = _N // _NW
_CH = 32
_NCHUNK = _PPT // _CH
_NL = 16


def _warp_body(img_hbm, tx_hbm, ty_hbm, out_hbm, tx_v, ty_v, idx_v, w_v,
               g_v, o_v, sem):
    wid = lax.axis_index("s") * 2 + lax.axis_index("c")
    base = wid * _PPT

    pltpu.sync_copy(tx_hbm.at[pl.ds(base, _PPT)], tx_v)
    pltpu.sync_copy(ty_hbm.at[pl.ds(base, _PPT)], ty_v)

    fone = jnp.float32(1.0)

    def chunk_body(g, carry):
        off = g * _CH
        for sub in range(_CH // _NL):
            s16 = off + sub * _NL
            p = base + s16 + lax.iota(jnp.int32, _NL)
            b = lax.div(p, _HW)
            r = p - b * _HW
            i = lax.div(r, _W)
            j = r - i * _W

            tx = tx_v[pl.ds(s16, _NL)]
            ty = ty_v[pl.ds(s16, _NL)]

            locx = jnp.clip(i.astype(jnp.float32) + tx, 0.0, float(_H - 1))
            x0 = jnp.minimum(locx.astype(jnp.int32), _H - 2)
            fx = locx - x0.astype(jnp.float32)

            locy = jnp.clip(j.astype(jnp.float32) + ty, 0.0, float(_W - 1))
            y0 = jnp.minimum(locy.astype(jnp.int32), _W - 2)
            fy = locy - y0.astype(jnp.float32)

            i00 = b * _HW + x0 * _W + y0
            gx = fone - fx
            gy = fone - fy

            idx_v[pl.ds(0 * _CH + s16 - off, _NL)] = i00
            idx_v[pl.ds(1 * _CH + s16 - off, _NL)] = i00 + 1
            idx_v[pl.ds(2 * _CH + s16 - off, _NL)] = i00 + _W
            idx_v[pl.ds(3 * _CH + s16 - off, _NL)] = i00 + _W + 1
            w_v[pl.ds(0 * _CH + s16 - off, _NL)] = gx * gy
            w_v[pl.ds(1 * _CH + s16 - off, _NL)] = gx * fy
            w_v[pl.ds(2 * _CH + s16 - off, _NL)] = fx * gy
            w_v[pl.ds(3 * _CH + s16 - off, _NL)] = fx * fy

        pltpu.async_copy(img_hbm.at[idx_v], g_v, sem).wait()

        def px_body(pp, c2):
            w00 = w_v[pl.ds(0 * _CH + pp, _NL)][0]
            w01 = w_v[pl.ds(1 * _CH + pp, _NL)][0]
            w10 = w_v[pl.ds(2 * _CH + pp, _NL)][0]
            w11 = w_v[pl.ds(3 * _CH + pp, _NL)][0]
            for c in range(_C // _NL):
                sl = pl.ds(c * _NL, _NL)
                o_v[pp, sl] = (w00 * g_v[0 * _CH + pp, sl]
                               + w01 * g_v[1 * _CH + pp, sl]
                               + w10 * g_v[2 * _CH + pp, sl]
                               + w11 * g_v[3 * _CH + pp, sl])
            return c2

        lax.fori_loop(0, _CH, px_body, 0, unroll=False)

        pltpu.sync_copy(o_v, out_hbm.at[pl.ds(base + off, _CH)])
        return carry

    lax.fori_loop(0, _NCHUNK, chunk_body, 0, unroll=False)


@jax.jit
def _warp_sc(img_flat, tx, ty):
    mesh = plsc.VectorSubcoreMesh(core_axis_name="c", subcore_axis_name="s")
    return pl.kernel(
        _warp_body,
        out_type=jax.ShapeDtypeStruct((_N, _C), jnp.float32),
        mesh=mesh,
        scratch_types=[
            pltpu.VMEM((_PPT,), jnp.float32),
            pltpu.VMEM((_PPT,), jnp.float32),
            pltpu.VMEM((4 * _CH,), jnp.int32),
            pltpu.VMEM((4 * _CH + _NL,), jnp.float32),
            pltpu.VMEM((4 * _CH, _C), jnp.float32),
            pltpu.VMEM((_CH, _C), jnp.float32),
            pltpu.SemaphoreType.DMA,
        ],
        compiler_params=pltpu.CompilerParams(use_tc_tiling_on_sc=False),
    )(img_flat, tx, ty)


def kernel(img, trf):
    B, H, W, C = img.shape
    img_flat = img.reshape(B * H * W, C)
    tx = trf[..., 0].reshape(-1)
    ty = trf[..., 1].reshape(-1)
    out = _warp_sc(img_flat, tx, ty)
    return out.reshape(B, H, W, C)


# speedup vs baseline: 1.0967x; 1.0791x over previous
import functools

import jax
import jax.numpy as jnp
from jax import lax
from jax.experimental import pallas as pl
from jax.experimental.pallas import tpu as pltpu
from jax.experimental.pallas import tpu_sc as plsc

_B, _H, _W, _C = 4, 384, 384, 96
_HW = _H * _W
_N = _B * _HW
_NW = 32
_PPT = _N // _NW
_CH = 32
_NCHUNK = _PPT // _CH
_NL = 16


def _warp_body(img_hbm, trf_hbm, out_hbm, tx_v, ty_v,
               idx_v, w_v, g_v, o_v, gsem, osem):
    wid = lax.axis_index("s") * 2 + lax.axis_index("c")
    base = wid * _PPT

    pltpu.sync_copy(trf_hbm.at[pl.ds(base, _PPT)], tx_v)
    pltpu.sync_copy(trf_hbm.at[pl.ds(_N + base, _PPT)], ty_v)

    fone = jnp.float32(1.0)

    def compute_idx(gg, slot):
        off = gg * _CH
        for h in range(_CH // _NL):
            s16 = off + h * _NL
            p = base + s16 + lax.iota(jnp.int32, _NL)
            b = lax.div(p, _HW)
            r = p - b * _HW
            i = lax.div(r, _W)
            j = r - i * _W

            tx = tx_v[pl.ds(s16, _NL)]
            ty = ty_v[pl.ds(s16, _NL)]

            locx = jnp.clip(i.astype(jnp.float32) + tx, 0.0, float(_H - 1))
            x0 = jnp.minimum(locx.astype(jnp.int32), _H - 2)
            fx = locx - x0.astype(jnp.float32)

            locy = jnp.clip(j.astype(jnp.float32) + ty, 0.0, float(_W - 1))
            y0 = jnp.minimum(locy.astype(jnp.int32), _W - 2)
            fy = locy - y0.astype(jnp.float32)

            i00 = b * _HW + x0 * _W + y0
            gx = fone - fx
            gy = fone - fy

            sl = pl.ds(h * _NL, _NL)
            idx_v[slot, pl.ds(0 * _CH + h * _NL, _NL)] = i00
            idx_v[slot, pl.ds(1 * _CH + h * _NL, _NL)] = i00 + 1
            idx_v[slot, pl.ds(2 * _CH + h * _NL, _NL)] = i00 + _W
            idx_v[slot, pl.ds(3 * _CH + h * _NL, _NL)] = i00 + _W + 1
            w_v[slot, pl.ds(0 * _CH + h * _NL, _NL)] = gx * gy
            w_v[slot, pl.ds(1 * _CH + h * _NL, _NL)] = gx * fy
            w_v[slot, pl.ds(2 * _CH + h * _NL, _NL)] = fx * gy
            w_v[slot, pl.ds(3 * _CH + h * _NL, _NL)] = fx * fy

    def start_gather(slot):
        pltpu.async_copy(img_hbm.at[idx_v.at[slot]], g_v.at[slot],
                         gsem.at[slot])

    def wait_gather(slot):
        pltpu.make_async_copy(img_hbm.at[idx_v.at[slot]], g_v.at[slot],
                              gsem.at[slot]).wait()

    def blend(slot):
        def px_body(pp, c2):
            w00 = w_v[slot, pl.ds(0 * _CH + pp, _NL)][0]
            w01 = w_v[slot, pl.ds(1 * _CH + pp, _NL)][0]
            w10 = w_v[slot, pl.ds(2 * _CH + pp, _NL)][0]
            w11 = w_v[slot, pl.ds(3 * _CH + pp, _NL)][0]
            for c in range(_C // _NL):
                sl = pl.ds(c * _NL, _NL)
                o_v[slot, pp, sl] = (w00 * g_v[slot, 0 * _CH + pp, sl]
                                     + w01 * g_v[slot, 1 * _CH + pp, sl]
                                     + w10 * g_v[slot, 2 * _CH + pp, sl]
                                     + w11 * g_v[slot, 3 * _CH + pp, sl])
            return c2

        lax.fori_loop(0, _CH, px_body, 0, unroll=False)

    def start_out(slot, gg):
        pltpu.async_copy(o_v.at[slot], out_hbm.at[pl.ds(base + gg * _CH, _CH)],
                         osem.at[slot])

    def wait_out(slot, gg):
        pltpu.make_async_copy(o_v.at[slot],
                              out_hbm.at[pl.ds(base + gg * _CH, _CH)],
                              osem.at[slot]).wait()

    compute_idx(0, 0)
    start_gather(0)

    def body(g, carry):
        slot = g & 1
        nslot = 1 - slot

        @pl.when(g + 1 < _NCHUNK)
        def _():
            compute_idx(g + 1, nslot)
            start_gather(nslot)

        wait_gather(slot)

        @pl.when(g >= 2)
        def _():
            wait_out(slot, g - 2)

        blend(slot)
        start_out(slot, g)
        return carry

    lax.fori_loop(0, _NCHUNK, body, 0, unroll=False)

    wait_out((_NCHUNK - 2) & 1, _NCHUNK - 2)
    wait_out((_NCHUNK - 1) & 1, _NCHUNK - 1)


@jax.jit
def _warp_sc(img_flat, trf_flat):
    mesh = plsc.VectorSubcoreMesh(core_axis_name="c", subcore_axis_name="s")
    return pl.kernel(
        _warp_body,
        out_type=jax.ShapeDtypeStruct((_N, _C), jnp.float32),
        mesh=mesh,
        scratch_types=[
            pltpu.VMEM((_PPT,), jnp.float32),
            pltpu.VMEM((_PPT,), jnp.float32),
            pltpu.VMEM((2, 4 * _CH), jnp.int32),
            pltpu.VMEM((2, 4 * _CH + _NL), jnp.float32),
            pltpu.VMEM((2, 4 * _CH, _C), jnp.float32),
            pltpu.VMEM((2, _CH, _C), jnp.float32),
            pltpu.SemaphoreType.DMA((2,)),
            pltpu.SemaphoreType.DMA((2,)),
        ],
        compiler_params=pltpu.CompilerParams(use_tc_tiling_on_sc=False),
    )(img_flat, trf_flat)


def kernel(img, trf):
    B, H, W, C = img.shape
    img_flat = img.reshape(B * H * W, C)
    txy = trf.reshape(B * H * W, 2).T.reshape(-1)
    out = _warp_sc(img_flat, txy)
    return out.reshape(B, H, W, C)


# speedup vs baseline: 1.2892x; 1.1755x over previous
import functools

import jax
import jax.numpy as jnp
from jax import lax
from jax.experimental import pallas as pl
from jax.experimental.pallas import tpu as pltpu
from jax.experimental.pallas import tpu_sc as plsc

_B, _H, _W, _C = 4, 384, 384, 96
_HW = _H * _W
_NW = 32
_PPT = _HW // _NW
_CH = 32
_NCHUNK = _PPT // _CH
_NL = 16


def _warp_body(img_hbm, trf_hbm, out_hbm, tx_v, ty_v,
               idx_v, w_v, g_v, o_v, gsem, osem):
    wid = lax.axis_index("s") * 2 + lax.axis_index("c")
    base = wid * _PPT

    pltpu.sync_copy(trf_hbm.at[pl.ds(base, _PPT)], tx_v)
    pltpu.sync_copy(trf_hbm.at[pl.ds(_HW + base, _PPT)], ty_v)

    fone = jnp.float32(1.0)

    def compute_idx(gg, slot):
        off = gg * _CH
        for h in range(_CH // _NL):
            s16 = off + h * _NL
            p = base + s16 + lax.iota(jnp.int32, _NL)
            i = lax.div(p, _W)
            j = p - i * _W

            tx = tx_v[pl.ds(s16, _NL)]
            ty = ty_v[pl.ds(s16, _NL)]

            locx = jnp.clip(i.astype(jnp.float32) + tx, 0.0, float(_H - 1))
            x0 = jnp.minimum(locx.astype(jnp.int32), _H - 2)
            fx = locx - x0.astype(jnp.float32)

            locy = jnp.clip(j.astype(jnp.float32) + ty, 0.0, float(_W - 1))
            y0 = jnp.minimum(locy.astype(jnp.int32), _W - 2)
            fy = locy - y0.astype(jnp.float32)

            i00 = x0 * _W + y0
            gx = fone - fx
            gy = fone - fy

            idx_v[slot, pl.ds(0 * _CH + h * _NL, _NL)] = i00
            idx_v[slot, pl.ds(1 * _CH + h * _NL, _NL)] = i00 + 1
            idx_v[slot, pl.ds(2 * _CH + h * _NL, _NL)] = i00 + _W
            idx_v[slot, pl.ds(3 * _CH + h * _NL, _NL)] = i00 + _W + 1
            w_v[slot, pl.ds(0 * _CH + h * _NL, _NL)] = gx * gy
            w_v[slot, pl.ds(1 * _CH + h * _NL, _NL)] = gx * fy
            w_v[slot, pl.ds(2 * _CH + h * _NL, _NL)] = fx * gy
            w_v[slot, pl.ds(3 * _CH + h * _NL, _NL)] = fx * fy

    def start_gather(slot):
        pltpu.async_copy(img_hbm.at[idx_v.at[slot]], g_v.at[slot],
                         gsem.at[slot])

    def wait_gather(slot):
        pltpu.make_async_copy(img_hbm.at[idx_v.at[slot]], g_v.at[slot],
                              gsem.at[slot]).wait()

    def blend(slot):
        def px_body(pp, c2):
            w00 = w_v[slot, pl.ds(0 * _CH + pp, _NL)][0]
            w01 = w_v[slot, pl.ds(1 * _CH + pp, _NL)][0]
            w10 = w_v[slot, pl.ds(2 * _CH + pp, _NL)][0]
            w11 = w_v[slot, pl.ds(3 * _CH + pp, _NL)][0]
            for c in range(_C // _NL):
                sl = pl.ds(c * _NL, _NL)
                o_v[slot, pp, sl] = (w00 * g_v[slot, 0 * _CH + pp, sl]
                                     + w01 * g_v[slot, 1 * _CH + pp, sl]
                                     + w10 * g_v[slot, 2 * _CH + pp, sl]
                                     + w11 * g_v[slot, 3 * _CH + pp, sl])
            return c2

        lax.fori_loop(0, _CH, px_body, 0, unroll=False)

    def start_out(slot, gg):
        pltpu.async_copy(o_v.at[slot], out_hbm.at[pl.ds(base + gg * _CH, _CH)],
                         osem.at[slot])

    def wait_out(slot, gg):
        pltpu.make_async_copy(o_v.at[slot],
                              out_hbm.at[pl.ds(base + gg * _CH, _CH)],
                              osem.at[slot]).wait()

    compute_idx(0, 0)
    start_gather(0)

    def body(g, carry):
        slot = g & 1
        nslot = 1 - slot

        @pl.when(g + 1 < _NCHUNK)
        def _():
            compute_idx(g + 1, nslot)
            start_gather(nslot)

        wait_gather(slot)

        @pl.when(g >= 2)
        def _():
            wait_out(slot, g - 2)

        blend(slot)
        start_out(slot, g)
        return carry

    lax.fori_loop(0, _NCHUNK, body, 0, unroll=False)

    wait_out((_NCHUNK - 2) & 1, _NCHUNK - 2)
    wait_out((_NCHUNK - 1) & 1, _NCHUNK - 1)


@jax.jit
def _warp_sc(img_flat, txy):
    mesh = plsc.VectorSubcoreMesh(core_axis_name="c", subcore_axis_name="s")
    return pl.kernel(
        _warp_body,
        out_type=jax.ShapeDtypeStruct((_HW, _C), jnp.float32),
        mesh=mesh,
        scratch_types=[
            pltpu.VMEM((_PPT,), jnp.float32),
            pltpu.VMEM((_PPT,), jnp.float32),
            pltpu.VMEM((2, 4 * _CH), jnp.int32),
            pltpu.VMEM((2, 4 * _CH + _NL), jnp.float32),
            pltpu.VMEM((2, 4 * _CH, _C), jnp.float32),
            pltpu.VMEM((2, _CH, _C), jnp.float32),
            pltpu.SemaphoreType.DMA((2,)),
            pltpu.SemaphoreType.DMA((2,)),
        ],
        compiler_params=pltpu.CompilerParams(use_tc_tiling_on_sc=False),
    )(img_flat, txy)


def kernel(img, trf):
    B, H, W, C = img.shape
    outs = []
    for b in range(B):
        img_b = img[b].reshape(H * W, C)
        txy_b = trf[b].reshape(H * W, 2).T.reshape(-1)
        outs.append(_warp_sc(img_b, txy_b))
    return jnp.stack(outs).reshape(B, H, W, C)
